# baseline (speedup 1.0000x reference)
import functools

import jax
import jax.numpy as jnp
from jax import lax
from jax.experimental import pallas as pl
from jax.experimental.pallas import tpu as pltpu
from jax.experimental.pallas import tpu_sc as plsc

_L = 16
_NS = 16
_NC = 2
_NW = _NC * _NS
_SH = 14


def _cdiv(a, b):
    return (a + b - 1) // b


def _matvec_body(x_ref, w_ref, c_ref, o_ref):
    w1 = jnp.sum(w_ref[0], axis=1, keepdims=True) * 2.0
    s = jnp.dot(x_ref[...], w1, preferred_element_type=jnp.float32)
    o_ref[...] = s / c_ref[...]


@functools.lru_cache(maxsize=None)
def _make_sc_kernel(N, E):
    CG = _cdiv(N, _NS * 128) * 128
    NP = _NS * CG
    E2 = E + N
    R4 = _cdiv(E2, _NW * 128)
    OP = _NW * 128 * R4
    C4 = R4 * 128

    mesh = plsc.VectorSubcoreMesh(core_axis_name="c", subcore_axis_name="s")

    def body(keys, ed_head, u_in, out,
             g_sh, u_v, g_v, keybuf3, edbuf, gbuf, keybuf4, resbuf, sem):
        cid = lax.axis_index("c")
        tid = lax.axis_index("s")
        wid = cid * _NS + tid

        d_k4 = pltpu.async_copy(keys.at[pl.ds(wid * C4, C4)], keybuf4, sem)
        d_u = pltpu.async_copy(u_in, u_v, sem)
        d_k3 = pltpu.async_copy(keys.at[pl.ds(tid * CG, CG)], keybuf3, sem)
        d_ed = pltpu.async_copy(ed_head.at[pl.ds(tid * CG, CG)], edbuf, sem)
        d_u.wait()
        d_k3.wait()
        d_ed.wait()
        d_k4.wait()

        def g_step(i, _):
            sl = pl.ds(i * _L, _L)
            srcv = jnp.right_shift(keybuf3[sl], _SH)
            uv = plsc.load_gather(u_v, [srcv])
            gbuf[sl] = uv / edbuf[sl]
            return _
        lax.fori_loop(0, CG // _L, g_step, None)
        pltpu.sync_copy(gbuf, g_sh.at[pl.ds(tid * CG, CG)])
        plsc.subcore_barrier()
        pltpu.sync_copy(g_sh, g_v)

        def p4_step(i, _):
            sl = pl.ds(i * _L, _L)
            k = keybuf4[sl]
            srcv = jnp.right_shift(k, _SH)
            dstv = jnp.bitwise_and(k, (1 << _SH) - 1)
            gs = plsc.load_gather(g_v, [srcv])
            gd = plsc.load_gather(g_v, [dstv])
            resbuf[sl] = gs - gd
            return _
        lax.fori_loop(0, C4 // _L, p4_step, None)
        pltpu.sync_copy(resbuf, out.at[pl.ds(wid * C4, C4)])

    return pl.kernel(
        body,
        out_type=jax.ShapeDtypeStruct((OP,), jnp.float32),
        mesh=mesh,
        scratch_types=[
            pltpu.VMEM_SHARED((NP,), jnp.float32),
            pltpu.VMEM((NP,), jnp.float32),
            pltpu.VMEM((NP,), jnp.float32),
            pltpu.VMEM((CG,), jnp.int32),
            pltpu.VMEM((CG,), jnp.float32),
            pltpu.VMEM((CG,), jnp.float32),
            pltpu.VMEM((C4,), jnp.int32),
            pltpu.VMEM((C4,), jnp.float32),
            pltpu.SemaphoreType.DMA,
        ],
        compiler_params=pltpu.CompilerParams(needs_layout_passes=False),
        name="gat2_sc",
    )


def kernel(x, edge_index, edge_dist, weight, attention):
    N, _ = x.shape
    E = edge_index.shape[0]
    h = weight.shape[2]

    src = edge_index[:, 0].astype(jnp.int32)
    dst = edge_index[:, 1].astype(jnp.int32)
    ed = edge_dist.astype(jnp.float32)

    keys = (src << _SH) | dst
    keys_s, ed_s = lax.sort([keys, ed], dimension=0, is_stable=True,
                            num_keys=1)

    qs = jnp.arange(N, dtype=jnp.int32) << _SH
    start = jnp.searchsorted(keys_s, qs, side="left").astype(jnp.int32)
    end = jnp.concatenate([start[1:], jnp.array([E], jnp.int32)])
    cnt = (end - start + 1).astype(jnp.float32)

    u_col = pl.pallas_call(
        _matvec_body,
        out_shape=jax.ShapeDtypeStruct((N, 1), jnp.float32),
        name="gat2_matvec",
    )(x, weight, cnt.reshape(N, 1))

    CG = _cdiv(N, _NS * 128) * 128
    NP = _NS * CG
    E2 = E + N
    R4 = _cdiv(E2, _NW * 128)
    OP = _NW * 128 * R4

    sent = jnp.int32(((NP - 1) << _SH) | (NP - 1))
    keys_pad = jnp.concatenate(
        [keys_s, jnp.full((OP - E,), sent, jnp.int32)])
    ed_head = jnp.concatenate(
        [ed_s[:N], jnp.ones((NP - N,), jnp.float32)])
    u_in = jnp.concatenate(
        [u_col[:, 0], jnp.zeros((NP - N,), jnp.float32)])

    out_pad = _make_sc_kernel(N, E)(keys_pad, ed_head, u_in)
    return out_pad[:E2].reshape(-1, h)


# speedup vs baseline: 1.5663x; 1.5663x over previous
import functools

import jax
import jax.numpy as jnp
from jax import lax
from jax.experimental import pallas as pl
from jax.experimental.pallas import tpu as pltpu
from jax.experimental.pallas import tpu_sc as plsc

_L = 16
_NS = 16
_NC = 2
_NW = _NC * _NS
_SH = 14


def _cdiv(a, b):
    return (a + b - 1) // b


def _matvec_body(x_ref, w_ref, o_ref):
    w1 = jnp.sum(w_ref[0], axis=1, keepdims=True) * 2.0
    o_ref[...] = jnp.dot(x_ref[...], w1, preferred_element_type=jnp.float32)


@functools.lru_cache(maxsize=None)
def _make_sc_kernel(N, E):
    CG = _cdiv(N, _NS * 128) * 128
    NP = _NS * CG
    KH = _cdiv(_cdiv(E, _NS * 128), 8) * 8
    E2 = E + N
    R4 = _cdiv(E2, _NW * 128)
    OP = _NW * 128 * R4
    C4 = R4 * 128
    GH = 12
    G1, REM1 = divmod(KH, GH)

    mesh = plsc.VectorSubcoreMesh(core_axis_name="c", subcore_axis_name="s")

    def body(src2d, keys, ed_head, s_in, out,
             cnt_sh, g_sh, u_v, g_v, s_v, cnt_v, srcbuf,
             keybuf3, edbuf, gbuf, keybuf4, resbuf, ones_v, zerobuf,
             sem, sem2):
        cid = lax.axis_index("c")
        tid = lax.axis_index("s")
        wid = cid * _NS + tid

        d_k4 = pltpu.async_copy(keys.at[pl.ds(wid * C4, C4)], keybuf4, sem2)

        def fill16(i, _):
            ones_v[pl.ds(i * _L, _L)] = jnp.ones((_L,), jnp.float32)
            return _
        lax.fori_loop(0, 128 // _L, fill16, None)

        def zero16(i, _):
            zerobuf[pl.ds(i * _L, _L)] = jnp.zeros((_L,), jnp.float32)
            return _
        lax.fori_loop(0, CG // _L, zero16, None)
        pltpu.sync_copy(zerobuf, cnt_sh.at[pl.ds(tid * CG, CG)])
        plsc.subcore_barrier()

        pltpu.sync_copy(src2d.at[pl.ds(tid * KH, KH)], srcbuf)

        def hist_group(gi, _):
            descs = []
            for r in range(GH):
                descs.append(pltpu.async_copy(
                    ones_v, cnt_sh.at[srcbuf.at[gi * GH + r]], sem, add=True))
            for d in descs:
                d.wait()
            return _
        lax.fori_loop(0, G1, hist_group, None)
        descs = []
        for r in range(REM1):
            descs.append(pltpu.async_copy(
                ones_v, cnt_sh.at[srcbuf.at[G1 * GH + r]], sem, add=True))
        for d in descs:
            d.wait()
        plsc.subcore_barrier()

        pltpu.sync_copy(cnt_sh, cnt_v)
        pltpu.sync_copy(s_in, s_v)

        def u_step(i, _):
            sl = pl.ds(i * _L, _L)
            u_v[sl] = s_v[sl] / (cnt_v[sl] + 1.0)
            return _
        lax.fori_loop(0, NP // _L, u_step, None)

        d_k3 = pltpu.async_copy(keys.at[pl.ds(tid * CG, CG)], keybuf3, sem)
        d_ed = pltpu.async_copy(ed_head.at[pl.ds(tid * CG, CG)], edbuf, sem)
        d_k3.wait()
        d_ed.wait()

        def g_step(i, _):
            sl = pl.ds(i * _L, _L)
            srcv = jnp.right_shift(keybuf3[sl], _SH)
            uv = plsc.load_gather(u_v, [srcv])
            gbuf[sl] = uv / edbuf[sl]
            return _
        lax.fori_loop(0, CG // _L, g_step, None)
        pltpu.sync_copy(gbuf, g_sh.at[pl.ds(tid * CG, CG)])
        plsc.subcore_barrier()
        pltpu.sync_copy(g_sh, g_v)
        d_k4.wait()

        def p4_step(i, _):
            sl = pl.ds(i * _L, _L)
            k = keybuf4[sl]
            srcv = jnp.right_shift(k, _SH)
            dstv = jnp.bitwise_and(k, (1 << _SH) - 1)
            gs = plsc.load_gather(g_v, [srcv])
            gd = plsc.load_gather(g_v, [dstv])
            resbuf[sl] = gs - gd
            return _
        lax.fori_loop(0, C4 // _L, p4_step, None)
        pltpu.sync_copy(resbuf, out.at[pl.ds(wid * C4, C4)])

    return pl.kernel(
        body,
        out_type=jax.ShapeDtypeStruct((OP,), jnp.float32),
        mesh=mesh,
        scratch_types=[
            pltpu.VMEM_SHARED((NP,), jnp.float32),
            pltpu.VMEM_SHARED((NP,), jnp.float32),
            pltpu.VMEM((NP,), jnp.float32),
            pltpu.VMEM((NP,), jnp.float32),
            pltpu.VMEM((NP,), jnp.float32),
            pltpu.VMEM((NP,), jnp.float32),
            pltpu.VMEM((KH, 128), jnp.int32),
            pltpu.VMEM((CG,), jnp.int32),
            pltpu.VMEM((CG,), jnp.float32),
            pltpu.VMEM((CG,), jnp.float32),
            pltpu.VMEM((C4,), jnp.int32),
            pltpu.VMEM((C4,), jnp.float32),
            pltpu.VMEM((128,), jnp.float32),
            pltpu.VMEM((CG,), jnp.float32),
            pltpu.SemaphoreType.DMA,
            pltpu.SemaphoreType.DMA,
        ],
        compiler_params=pltpu.CompilerParams(needs_layout_passes=False),
        name="gat2_sc",
    )


def kernel(x, edge_index, edge_dist, weight, attention):
    N, _ = x.shape
    E = edge_index.shape[0]
    h = weight.shape[2]

    src = edge_index[:, 0].astype(jnp.int32)
    dst = edge_index[:, 1].astype(jnp.int32)
    ed = edge_dist.astype(jnp.float32)

    keys = (src << _SH) | dst
    keys_s, ed_s = lax.sort([keys, ed], dimension=0, is_stable=True,
                            num_keys=1)

    CG = _cdiv(N, _NS * 128) * 128
    NP = _NS * CG
    KH = _cdiv(_cdiv(E, _NS * 128), 8) * 8
    EP = _NS * 128 * KH
    E2 = E + N
    R4 = _cdiv(E2, _NW * 128)
    OP = _NW * 128 * R4

    pad_node = jnp.int32(NP - 1)
    src2d = jnp.concatenate(
        [src, jnp.full((EP - E,), pad_node, jnp.int32)]).reshape(EP // 128, 128)

    sent = jnp.int32(((NP - 1) << _SH) | (NP - 1))
    keys_pad = jnp.concatenate(
        [keys_s, jnp.full((OP - E,), sent, jnp.int32)])
    ed_head = jnp.concatenate(
        [ed_s[:N], jnp.ones((NP - N,), jnp.float32)])

    s_col = pl.pallas_call(
        _matvec_body,
        out_shape=jax.ShapeDtypeStruct((N, 1), jnp.float32),
        name="gat2_matvec",
    )(x, weight)
    s_in = jnp.concatenate([s_col[:, 0], jnp.zeros((NP - N,), jnp.float32)])

    out_pad = _make_sc_kernel(N, E)(src2d, keys_pad, ed_head, s_in)
    return out_pad[:E2].reshape(-1, h)


# speedup vs baseline: 1.7932x; 1.1448x over previous
import functools

import jax
import jax.numpy as jnp
from jax import lax
from jax.experimental import pallas as pl
from jax.experimental.pallas import tpu as pltpu
from jax.experimental.pallas import tpu_sc as plsc

_L = 16
_NS = 16
_NC = 2
_NW = _NC * _NS
_SH = 14


def _cdiv(a, b):
    return (a + b - 1) // b


def _matvec_body(x_ref, w_ref, o_ref):
    w1 = jnp.sum(w_ref[0], axis=1, keepdims=True) * 2.0
    o_ref[...] = jnp.dot(x_ref[...], w1, preferred_element_type=jnp.float32)


@functools.lru_cache(maxsize=None)
def _make_sc_kernel(N, E):
    CG = _cdiv(N, _NS * 128) * 128
    NP = _NS * CG
    KH = _cdiv(_cdiv(E, _NS * 128), 8) * 8
    E2 = E + N
    R4 = _cdiv(E2, _NW * 128)
    OP = _NW * 128 * R4
    C4 = R4 * 128
    GH = 12
    G1, REM1 = divmod(KH, GH)

    mesh = plsc.VectorSubcoreMesh(core_axis_name="c", subcore_axis_name="s")

    def body(src2d, keys, ed_head, s_in, out,
             cnt_sh, g_sh, u_v, g_v, s_v, cnt_v, srcbuf,
             keybuf3, edbuf, gbuf, keybuf4, resbuf, ones_v, zerobuf,
             sem, sem2):
        cid = lax.axis_index("c")
        tid = lax.axis_index("s")
        wid = cid * _NS + tid

        d_k4 = pltpu.async_copy(keys.at[pl.ds(wid * C4, C4)], keybuf4, sem2)

        def fill16(i, _):
            ones_v[pl.ds(i * _L, _L)] = jnp.ones((_L,), jnp.float32)
            return _
        lax.fori_loop(0, 128 // _L, fill16, None)

        def zero16(i, _):
            zerobuf[pl.ds(i * _L, _L)] = jnp.zeros((_L,), jnp.float32)
            return _
        lax.fori_loop(0, CG // _L, zero16, None)
        pltpu.sync_copy(zerobuf, cnt_sh.at[pl.ds(tid * CG, CG)])
        plsc.subcore_barrier()

        pltpu.sync_copy(src2d.at[pl.ds(tid * KH, KH)], srcbuf)

        def hist_group(gi, _):
            descs = []
            for r in range(GH):
                descs.append(pltpu.async_copy(
                    ones_v, cnt_sh.at[srcbuf.at[gi * GH + r]], sem, add=True))
            for d in descs:
                d.wait()
            return _
        lax.fori_loop(0, G1, hist_group, None)
        descs = []
        for r in range(REM1):
            descs.append(pltpu.async_copy(
                ones_v, cnt_sh.at[srcbuf.at[G1 * GH + r]], sem, add=True))
        for d in descs:
            d.wait()
        plsc.subcore_barrier()

        pltpu.sync_copy(cnt_sh, cnt_v)
        pltpu.sync_copy(s_in, s_v)

        def u_step(i, _):
            sl = pl.ds(i * _L, _L)
            u_v[sl] = s_v[sl] / (cnt_v[sl] + 1.0)
            return _
        lax.fori_loop(0, NP // _L, u_step, None)

        d_k3 = pltpu.async_copy(keys.at[pl.ds(tid * CG, CG)], keybuf3, sem)
        d_ed = pltpu.async_copy(ed_head.at[pl.ds(tid * CG, CG)], edbuf, sem)
        d_k3.wait()
        d_ed.wait()

        def g_step(i, _):
            sl = pl.ds(i * _L, _L)
            srcv = jnp.right_shift(keybuf3[sl], _SH)
            uv = plsc.load_gather(u_v, [srcv])
            gbuf[sl] = uv / edbuf[sl]
            return _
        lax.fori_loop(0, CG // _L, g_step, None)
        pltpu.sync_copy(gbuf, g_sh.at[pl.ds(tid * CG, CG)])
        plsc.subcore_barrier()
        pltpu.sync_copy(g_sh, g_v)
        d_k4.wait()

        def p4_step(i, _):
            sl = pl.ds(i * _L, _L)
            k = keybuf4[sl]
            srcv = jnp.right_shift(k, _SH)
            dstv = jnp.bitwise_and(k, (1 << _SH) - 1)
            gs = plsc.load_gather(g_v, [srcv])
            gd = plsc.load_gather(g_v, [dstv])
            resbuf[sl] = gs - gd
            return _
        lax.fori_loop(0, C4 // _L, p4_step, None)
        pltpu.sync_copy(resbuf, out.at[pl.ds(wid * C4, C4)])

    return pl.kernel(
        body,
        out_type=jax.ShapeDtypeStruct((OP,), jnp.float32),
        mesh=mesh,
        scratch_types=[
            pltpu.VMEM_SHARED((NP,), jnp.float32),
            pltpu.VMEM_SHARED((NP,), jnp.float32),
            pltpu.VMEM((NP,), jnp.float32),
            pltpu.VMEM((NP,), jnp.float32),
            pltpu.VMEM((NP,), jnp.float32),
            pltpu.VMEM((NP,), jnp.float32),
            pltpu.VMEM((KH, 128), jnp.int32),
            pltpu.VMEM((CG,), jnp.int32),
            pltpu.VMEM((CG,), jnp.float32),
            pltpu.VMEM((CG,), jnp.float32),
            pltpu.VMEM((C4,), jnp.int32),
            pltpu.VMEM((C4,), jnp.float32),
            pltpu.VMEM((128,), jnp.float32),
            pltpu.VMEM((CG,), jnp.float32),
            pltpu.SemaphoreType.DMA,
            pltpu.SemaphoreType.DMA,
        ],
        compiler_params=pltpu.CompilerParams(needs_layout_passes=False),
        name="gat2_sc",
    )


def kernel(x, edge_index, edge_dist, weight, attention):
    N, _ = x.shape
    E = edge_index.shape[0]
    h = weight.shape[2]

    src = edge_index[:, 0].astype(jnp.int32)
    dst = edge_index[:, 1].astype(jnp.int32)
    ed = edge_dist.astype(jnp.float32)

    keys = (src << _SH) | dst
    iota = lax.iota(jnp.int32, E)
    keys_s, perm = lax.sort([keys, iota], dimension=0, is_stable=False,
                            num_keys=2)
    ed_s = ed[perm[:x.shape[0]]]

    CG = _cdiv(N, _NS * 128) * 128
    NP = _NS * CG
    KH = _cdiv(_cdiv(E, _NS * 128), 8) * 8
    EP = _NS * 128 * KH
    E2 = E + N
    R4 = _cdiv(E2, _NW * 128)
    OP = _NW * 128 * R4

    pad_node = jnp.int32(NP - 1)
    src2d = jnp.concatenate(
        [src, jnp.full((EP - E,), pad_node, jnp.int32)]).reshape(EP // 128, 128)

    sent = jnp.int32(((NP - 1) << _SH) | (NP - 1))
    keys_pad = jnp.concatenate(
        [keys_s, jnp.full((OP - E,), sent, jnp.int32)])
    ed_head = jnp.concatenate(
        [ed_s[:N], jnp.ones((NP - N,), jnp.float32)])

    s_col = pl.pallas_call(
        _matvec_body,
        out_shape=jax.ShapeDtypeStruct((N, 1), jnp.float32),
        name="gat2_matvec",
    )(x, weight)
    s_in = jnp.concatenate([s_col[:, 0], jnp.zeros((NP - N,), jnp.float32)])

    out_pad = _make_sc_kernel(N, E)(src2d, keys_pad, ed_head, s_in)
    return out_pad[:E2].reshape(-1, h)


# speedup vs baseline: 1.8026x; 1.0053x over previous
import functools

import jax
import jax.numpy as jnp
from jax import lax
from jax.experimental import pallas as pl
from jax.experimental.pallas import tpu as pltpu
from jax.experimental.pallas import tpu_sc as plsc

_L = 16
_NS = 16
_NC = 2
_NW = _NC * _NS
_SH = 14


def _cdiv(a, b):
    return (a + b - 1) // b


def _matvec_body(x_ref, w_ref, o_ref):
    w1 = jnp.sum(w_ref[0], axis=1, keepdims=True) * 2.0
    o_ref[...] = jnp.dot(x_ref[...], w1, preferred_element_type=jnp.float32)


@functools.lru_cache(maxsize=None)
def _make_hist_kernel(N, E):
    CG = _cdiv(N, _NS * 128) * 128
    NP = _NS * CG
    KH2 = _cdiv(_cdiv(E, _NW * 128), 4) * 4
    GH = 12
    G1, REM1 = divmod(KH2, GH)

    mesh = plsc.VectorSubcoreMesh(core_axis_name="c", subcore_axis_name="s")

    def body(src2d, out, cnt_sh, srcbuf, ones_v, zerobuf, sem):
        cid = lax.axis_index("c")
        tid = lax.axis_index("s")
        wid = cid * _NS + tid

        def fill16(i, _):
            ones_v[pl.ds(i * _L, _L)] = jnp.ones((_L,), jnp.float32)
            return _
        lax.fori_loop(0, 128 // _L, fill16, None)

        def zero16(i, _):
            zerobuf[pl.ds(i * _L, _L)] = jnp.zeros((_L,), jnp.float32)
            return _
        lax.fori_loop(0, CG // _L, zero16, None)
        pltpu.sync_copy(zerobuf, cnt_sh.at[pl.ds(tid * CG, CG)])
        plsc.subcore_barrier()

        pltpu.sync_copy(src2d.at[pl.ds(wid * KH2, KH2)], srcbuf)

        def hist_group(gi, _):
            descs = []
            for r in range(GH):
                descs.append(pltpu.async_copy(
                    ones_v, cnt_sh.at[srcbuf.at[gi * GH + r]], sem, add=True))
            for d in descs:
                d.wait()
            return _
        lax.fori_loop(0, G1, hist_group, None)
        descs = []
        for r in range(REM1):
            descs.append(pltpu.async_copy(
                ones_v, cnt_sh.at[srcbuf.at[G1 * GH + r]], sem, add=True))
        for d in descs:
            d.wait()
        plsc.subcore_barrier()

        pltpu.sync_copy(cnt_sh.at[pl.ds(tid * CG, CG)],
                        out.at[pl.ds(cid * NP + tid * CG, CG)])

    return pl.kernel(
        body,
        out_type=jax.ShapeDtypeStruct((_NC * NP,), jnp.float32),
        mesh=mesh,
        scratch_types=[
            pltpu.VMEM_SHARED((NP,), jnp.float32),
            pltpu.VMEM((KH2, 128), jnp.int32),
            pltpu.VMEM((128,), jnp.float32),
            pltpu.VMEM((CG,), jnp.float32),
            pltpu.SemaphoreType.DMA,
        ],
        compiler_params=pltpu.CompilerParams(needs_layout_passes=False),
        name="gat2_sc_hist",
    )


@functools.lru_cache(maxsize=None)
def _make_main_kernel(N, E):
    CG = _cdiv(N, _NS * 128) * 128
    NP = _NS * CG
    E2 = E + N
    R4 = _cdiv(E2, _NW * 128)
    OP = _NW * 128 * R4
    C4 = R4 * 128

    mesh = plsc.VectorSubcoreMesh(core_axis_name="c", subcore_axis_name="s")

    def body(keys, ed_head, cnt_in, s_in, out,
             g_sh, u_v, g_v, s_v, cnt_v, keybuf3, edbuf, gbuf,
             keybuf4, resbuf, sem, sem2):
        cid = lax.axis_index("c")
        tid = lax.axis_index("s")
        wid = cid * _NS + tid

        d_k3 = pltpu.async_copy(keys.at[pl.ds(tid * CG, CG)], keybuf3, sem2)
        d_ed = pltpu.async_copy(ed_head.at[pl.ds(tid * CG, CG)], edbuf, sem2)
        d_k4 = pltpu.async_copy(keys.at[pl.ds(wid * C4, C4)], keybuf4, sem2)

        d_c = pltpu.async_copy(cnt_in, cnt_v, sem)
        d_s = pltpu.async_copy(s_in, s_v, sem)
        d_c.wait()
        d_s.wait()

        def u_step(i, _):
            sl = pl.ds(i * _L, _L)
            u_v[sl] = s_v[sl] / (cnt_v[sl] + 1.0)
            return _
        lax.fori_loop(0, NP // _L, u_step, None)

        d_k3.wait()
        d_ed.wait()
        d_k4.wait()

        def g_step(i, _):
            sl = pl.ds(i * _L, _L)
            srcv = jnp.right_shift(keybuf3[sl], _SH)
            uv = plsc.load_gather(u_v, [srcv])
            gbuf[sl] = uv / edbuf[sl]
            return _
        lax.fori_loop(0, CG // _L, g_step, None)
        pltpu.sync_copy(gbuf, g_sh.at[pl.ds(tid * CG, CG)])
        plsc.subcore_barrier()
        pltpu.sync_copy(g_sh, g_v)

        def p4_step(i, _):
            sl = pl.ds(i * _L, _L)
            k = keybuf4[sl]
            srcv = jnp.right_shift(k, _SH)
            dstv = jnp.bitwise_and(k, (1 << _SH) - 1)
            gs = plsc.load_gather(g_v, [srcv])
            gd = plsc.load_gather(g_v, [dstv])
            resbuf[sl] = gs - gd
            return _
        lax.fori_loop(0, C4 // _L, p4_step, None)
        pltpu.sync_copy(resbuf, out.at[pl.ds(wid * C4, C4)])

    return pl.kernel(
        body,
        out_type=jax.ShapeDtypeStruct((OP,), jnp.float32),
        mesh=mesh,
        scratch_types=[
            pltpu.VMEM_SHARED((NP,), jnp.float32),
            pltpu.VMEM((NP,), jnp.float32),
            pltpu.VMEM((NP,), jnp.float32),
            pltpu.VMEM((NP,), jnp.float32),
            pltpu.VMEM((NP,), jnp.float32),
            pltpu.VMEM((CG,), jnp.int32),
            pltpu.VMEM((CG,), jnp.float32),
            pltpu.VMEM((CG,), jnp.float32),
            pltpu.VMEM((C4,), jnp.int32),
            pltpu.VMEM((C4,), jnp.float32),
            pltpu.SemaphoreType.DMA,
            pltpu.SemaphoreType.DMA,
        ],
        compiler_params=pltpu.CompilerParams(needs_layout_passes=False),
        name="gat2_sc_main",
    )


def kernel(x, edge_index, edge_dist, weight, attention):
    N, _ = x.shape
    E = edge_index.shape[0]
    h = weight.shape[2]

    src = edge_index[:, 0].astype(jnp.int32)
    dst = edge_index[:, 1].astype(jnp.int32)
    ed = edge_dist.astype(jnp.float32)

    CG = _cdiv(N, _NS * 128) * 128
    NP = _NS * CG
    KH2 = _cdiv(_cdiv(E, _NW * 128), 4) * 4
    EP = _NW * 128 * KH2
    E2 = E + N
    R4 = _cdiv(E2, _NW * 128)
    OP = _NW * 128 * R4

    pad_node = jnp.int32(NP - 1)
    src2d = jnp.concatenate(
        [src, jnp.full((EP - E,), pad_node, jnp.int32)]).reshape(EP // 128, 128)
    cnt2 = _make_hist_kernel(N, E)(src2d)
    cnt = cnt2[:NP] + cnt2[NP:]

    keys = (src << _SH) | dst
    iota = lax.iota(jnp.int32, E)
    keys_s, perm = lax.sort([keys, iota], dimension=0, is_stable=False,
                            num_keys=2)
    ed_s = ed[perm[:N]]

    sent = jnp.int32(((NP - 1) << _SH) | (NP - 1))
    keys_pad = jnp.concatenate(
        [keys_s, jnp.full((OP - E,), sent, jnp.int32)])
    ed_head = jnp.concatenate([ed_s, jnp.ones((NP - N,), jnp.float32)])

    s_col = pl.pallas_call(
        _matvec_body,
        out_shape=jax.ShapeDtypeStruct((N, 1), jnp.float32),
        name="gat2_matvec",
    )(x, weight)
    s_in = jnp.concatenate([s_col[:, 0], jnp.zeros((NP - N,), jnp.float32)])

    out_pad = _make_main_kernel(N, E)(keys_pad, ed_head, cnt, s_in)
    return out_pad[:E2].reshape(-1, h)
